# baseline (speedup 1.0000x reference)
import functools

import jax
import jax.numpy as jnp
from jax import lax
from jax.experimental import pallas as pl
from jax.experimental.pallas import tpu as pltpu
from jax.experimental.pallas import tpu_sc as plsc


def kernel(x, continuous_x_res, tables):
    B, C = x.shape
    _, NCONT, D = continuous_x_res.shape
    V = tables.shape[1]
    OUT_C = C + NCONT

    tables_flat = tables.reshape(C * V, D)
    flat_idx = x + (jnp.arange(C, dtype=jnp.int32) * V)[None, :]

    NC, NS = 2, 16
    NW = NC * NS
    b_per_w = B // NW
    BK = 16
    steps = b_per_w // BK

    mesh = plsc.VectorSubcoreMesh(core_axis_name="c", subcore_axis_name="s")

    @functools.partial(
        pl.kernel,
        mesh=mesh,
        out_type=jax.ShapeDtypeStruct((B, OUT_C, D), jnp.float32),
        compiler_params=pltpu.CompilerParams(use_tc_tiling_on_sc=False),
        scratch_types=[
            pltpu.VMEM((BK, C), jnp.int32),
            pltpu.VMEM((BK, OUT_C, D), jnp.float32),
            pltpu.SemaphoreType.DMA,
            pltpu.SemaphoreType.DMA,
        ],
    )
    def k(tab_hbm, idx_hbm, cont_hbm, out_hbm, idx_v, vbuf, sem_g, sem_c):
        wid = lax.axis_index("s") * NC + lax.axis_index("c")
        base = wid * b_per_w

        @pl.loop(0, steps)
        def _(t):
            row0 = base + t * BK
            pltpu.sync_copy(idx_hbm.at[pl.ds(row0, BK)], idx_v)
            cont_cp = pltpu.async_copy(
                cont_hbm.at[pl.ds(row0, BK)],
                vbuf.at[:, pl.ds(C, NCONT)],
                sem_c,
            )
            gathers = []
            for j in range(BK):
                gathers.append(pltpu.async_copy(
                    tab_hbm.at[idx_v.at[j]],
                    vbuf.at[j, pl.ds(0, C)],
                    sem_g,
                ))
            for cp in gathers:
                cp.wait()
            cont_cp.wait()
            pltpu.sync_copy(vbuf, out_hbm.at[pl.ds(row0, BK)])

    return k(tables_flat, flat_idx, continuous_x_res)
